# baseline (speedup 1.0000x reference)
import functools

import jax
import jax.numpy as jnp
from jax import lax
from jax.experimental import pallas as pl
from jax.experimental.pallas import tpu as pltpu
from jax.experimental.pallas import tpu_sc as plsc

H = 128
_EB = 2000
_NB = 2000
_NC = 2
_NS = 16
_NW = _NC * _NS
_NP = 10240


def _slog(x):
    return jnp.sign(x) * jnp.log1p(jnp.abs(x))


def _silu(x):
    return x * jax.nn.sigmoid(x)


def _pre_body(h_ref, smr_ref, w1s_ref, w1d_ref, ws_ref, wd_ref, wp_ref,
              asrc_ref, adst_ref, pre_ref):
    h = h_ref[...]
    smr = smr_ref[...]
    sl = _slog(smr[:, 0:4])
    f = jnp.concatenate([sl, smr[:, 4:16]], axis=-1)
    asrc_ref[...] = (jnp.dot(h, w1s_ref[...], preferred_element_type=jnp.float32)
                     + jnp.dot(f, ws_ref[...], preferred_element_type=jnp.float32))
    adst_ref[...] = (jnp.dot(h, w1d_ref[...], preferred_element_type=jnp.float32)
                     + jnp.dot(f, wd_ref[...], preferred_element_type=jnp.float32))
    pre_ref[...] = jnp.dot(f, wp_ref[...], preferred_element_type=jnp.float32)


def _pre_kernel(h, smr, w1s, w1d, ws, wd, wp):
    N = h.shape[0]
    grid = N // _NB
    row = lambda i: (i, 0)
    full = lambda i: (0, 0)
    out = jax.ShapeDtypeStruct((N, H), jnp.float32)
    return pl.pallas_call(
        _pre_body,
        grid=(grid,),
        in_specs=[
            pl.BlockSpec((_NB, H), row),
            pl.BlockSpec((_NB, 16), row),
            pl.BlockSpec((H, H), full),
            pl.BlockSpec((H, H), full),
            pl.BlockSpec((16, H), full),
            pl.BlockSpec((16, H), full),
            pl.BlockSpec((16, H), full),
        ],
        out_specs=[pl.BlockSpec((_NB, H), row)] * 3,
        out_shape=[out, out, out],
    )(h, smr, w1s, w1d, ws, wd, wp)


def _make_ring_gather(E2, width, gc, untiled):
    ew = E2 // _NW
    nb = ew // gc
    npairs = nb // 2
    has_tail = (nb % 2) == 1
    mesh = plsc.VectorSubcoreMesh(core_axis_name="c", subcore_axis_name="s")
    kw = {}
    if untiled:
        kw["compiler_params"] = pltpu.CompilerParams(use_tc_tiling_on_sc=False)

    @functools.partial(
        pl.kernel,
        out_type=[
            jax.ShapeDtypeStruct((E2, width), jnp.float32),
            jax.ShapeDtypeStruct((E2, width), jnp.float32),
        ],
        mesh=mesh,
        scratch_types=(
            [pltpu.VMEM((gc,), jnp.int32)] * 4
            + [pltpu.VMEM((gc, width), jnp.float32)] * 4
            + [pltpu.SemaphoreType.DMA] * 4
        ),
        **kw,
    )
    def gk(src_h, dst_h, ta_h, tb_h, ga_h, gb_h,
           ia0, ia1, id0, id1, ra0, ra1, rb0, rb1, sa0, sa1, sb0, sb1):
        wid = lax.axis_index("s") * _NC + lax.axis_index("c")
        base = wid * ew

        pltpu.sync_copy(src_h.at[pl.ds(base, gc)], ia0)
        pltpu.sync_copy(dst_h.at[pl.ds(base, gc)], id0)
        pltpu.async_copy(ta_h.at[ia0], ra0, sa0)
        pltpu.async_copy(tb_h.at[id0], rb0, sb0)

        def pair(j, _):
            c0 = base + (2 * j) * gc
            c1 = c0 + gc
            pltpu.sync_copy(src_h.at[pl.ds(c1, gc)], ia1)
            pltpu.sync_copy(dst_h.at[pl.ds(c1, gc)], id1)
            pltpu.async_copy(ta_h.at[ia1], ra1, sa1)
            pltpu.async_copy(tb_h.at[id1], rb1, sb1)
            pltpu.make_async_copy(ta_h.at[ia0], ra0, sa0).wait()
            pltpu.sync_copy(ra0, ga_h.at[pl.ds(c0, gc)])
            pltpu.make_async_copy(tb_h.at[id0], rb0, sb0).wait()
            pltpu.sync_copy(rb0, gb_h.at[pl.ds(c0, gc)])

            @pl.when(2 * j + 2 < nb)
            def _():
                c2 = c1 + gc
                pltpu.sync_copy(src_h.at[pl.ds(c2, gc)], ia0)
                pltpu.sync_copy(dst_h.at[pl.ds(c2, gc)], id0)
                pltpu.async_copy(ta_h.at[ia0], ra0, sa0)
                pltpu.async_copy(tb_h.at[id0], rb0, sb0)

            pltpu.make_async_copy(ta_h.at[ia1], ra1, sa1).wait()
            pltpu.sync_copy(ra1, ga_h.at[pl.ds(c1, gc)])
            pltpu.make_async_copy(tb_h.at[id1], rb1, sb1).wait()
            pltpu.sync_copy(rb1, gb_h.at[pl.ds(c1, gc)])
            return 0

        lax.fori_loop(0, npairs, pair, 0)

        if has_tail:
            ct = base + (nb - 1) * gc
            pltpu.make_async_copy(ta_h.at[ia0], ra0, sa0).wait()
            pltpu.sync_copy(ra0, ga_h.at[pl.ds(ct, gc)])
            pltpu.make_async_copy(tb_h.at[id0], rb0, sb0).wait()
            pltpu.sync_copy(rb0, gb_h.at[pl.ds(ct, gc)])

    return gk


def _make_scatter(E2, sc_chunk):
    half = E2 // _NC
    ew = half // _NS
    nb = ew // sc_chunk
    npairs = nb // 2
    has_tail = (nb % 2) == 1
    rows_per_tile = _NP // _NS
    mesh = plsc.VectorSubcoreMesh(core_axis_name="c", subcore_axis_name="s")

    @functools.partial(
        pl.kernel,
        out_type=jax.ShapeDtypeStruct((_NC, _NP, H), jnp.float32),
        mesh=mesh,
        scratch_types=(
            [pltpu.VMEM((sc_chunk,), jnp.int32)]
            + [pltpu.VMEM((sc_chunk, H), jnp.float32)]
            + [pltpu.VMEM_SHARED((_NP, H), jnp.float32)]
        ),
    )
    def sk(m_h, dst_h, z_h, out_h, i0, r0, agg_s):
        core = lax.axis_index("c")
        sid = lax.axis_index("s")
        rbase = sid * rows_per_tile
        pltpu.sync_copy(z_h.at[pl.ds(rbase, rows_per_tile)],
                        agg_s.at[pl.ds(rbase, rows_per_tile)])
        plsc.subcore_barrier()
        base = core * half + sid * ew

        def chunk(j, _):
            c0 = base + j * sc_chunk
            pltpu.sync_copy(dst_h.at[pl.ds(c0, sc_chunk)], i0)
            pltpu.sync_copy(m_h.at[pl.ds(c0, sc_chunk)], r0)
            pltpu.sync_copy(r0, agg_s.at[i0], add=True)
            return 0

        lax.fori_loop(0, nb, chunk, 0)

        plsc.subcore_barrier()
        pltpu.sync_copy(agg_s.at[pl.ds(rbase, rows_per_tile)],
                        out_h.at[core, pl.ds(rbase, rows_per_tile)])

    return sk


def _edge_body(ga_ref, gb_ref, ss_ref, sd_ref, rf_ref, cf_ref, w24_ref,
               ew2_ref, eb2_ref, out_ref):
    s = ss_ref[:, 0:4]
    d = sd_ref[:, 0:4]
    sm = s + d
    df = d - s
    stats = jnp.concatenate([
        jnp.sum(s * d, -1, keepdims=True),
        jnp.sum(df * df, -1, keepdims=True),
        jnp.sum(s * s, -1, keepdims=True),
        jnp.sum(d * d, -1, keepdims=True)], -1)
    B = s.shape[0]
    ohr = (rf_ref[...] == lax.broadcasted_iota(jnp.int32, (B, 8), 1)
           .astype(jnp.float32)).astype(jnp.float32)
    ohc = (cf_ref[...] == lax.broadcasted_iota(jnp.int32, (B, 4), 1)
           .astype(jnp.float32)).astype(jnp.float32)
    feats = jnp.concatenate(
        [_slog(sm), _slog(jnp.abs(df)), _slog(stats), ohr, ohc], -1)
    x = (ga_ref[...] + gb_ref[...]
         + jnp.dot(feats, w24_ref[...], preferred_element_type=jnp.float32))
    a = _silu(x)
    out_ref[...] = _silu(
        jnp.dot(a, ew2_ref[...], preferred_element_type=jnp.float32)
        + eb2_ref[...])


def _edge_mlp(ga, gb, ss, sd, relf, chf, w24, ew2, eb2):
    E2 = ga.shape[0]
    grid = E2 // _EB
    row = lambda i: (i, 0)
    full = lambda i: (0, 0)
    return pl.pallas_call(
        _edge_body,
        grid=(grid,),
        in_specs=[
            pl.BlockSpec((_EB, H), row),
            pl.BlockSpec((_EB, H), row),
            pl.BlockSpec((_EB, 16), row),
            pl.BlockSpec((_EB, 16), row),
            pl.BlockSpec((_EB, 1), row),
            pl.BlockSpec((_EB, 1), row),
            pl.BlockSpec((24, H), full),
            pl.BlockSpec((H, H), full),
            pl.BlockSpec((1, H), full),
        ],
        out_specs=pl.BlockSpec((_EB, H), row),
        out_shape=jax.ShapeDtypeStruct((E2, H), jnp.float32),
    )(ga, gb, ss, sd, relf, chf, w24, ew2, eb2.reshape(1, H))


def _node_body(h_ref, p0_ref, p1_ref, p2_ref, p3_ref, pre_ref, w1h_ref,
               w1a_ref, w2_ref, nb2_ref, g_ref, b_ref, out_ref):
    agg = ((p0_ref[0] + p1_ref[0]) + (p2_ref[0] + p3_ref[0]))
    x = _silu(jnp.dot(h_ref[...], w1h_ref[...],
                      preferred_element_type=jnp.float32)
              + jnp.dot(agg, w1a_ref[...], preferred_element_type=jnp.float32)
              + pre_ref[...])
    y = h_ref[...] + jnp.dot(x, w2_ref[...],
                             preferred_element_type=jnp.float32) + nb2_ref[...]
    mu = jnp.mean(y, -1, keepdims=True)
    yc = y - mu
    var = jnp.mean(yc * yc, -1, keepdims=True)
    out_ref[...] = yc * jax.lax.rsqrt(var + 1e-5) * g_ref[...] + b_ref[...]


def _node_mlp(h, parts1, parts2, pre, w1h, w1a, w2, nb2, g, b):
    N = h.shape[0]
    grid = N // _NB
    full = lambda i: (0, 0)
    row = lambda i: (i, 0)
    c0 = lambda i: (0, i, 0)
    c1 = lambda i: (1, i, 0)
    return pl.pallas_call(
        _node_body,
        grid=(grid,),
        in_specs=[
            pl.BlockSpec((_NB, H), row),
            pl.BlockSpec((1, _NB, H), c0),
            pl.BlockSpec((1, _NB, H), c1),
            pl.BlockSpec((1, _NB, H), c0),
            pl.BlockSpec((1, _NB, H), c1),
            pl.BlockSpec((_NB, H), row),
            pl.BlockSpec((H, H), full),
            pl.BlockSpec((H, H), full),
            pl.BlockSpec((H, H), full),
            pl.BlockSpec((1, H), full),
            pl.BlockSpec((1, H), full),
            pl.BlockSpec((1, H), full),
        ],
        out_specs=pl.BlockSpec((_NB, H), row),
        out_shape=jax.ShapeDtypeStruct((N, H), jnp.float32),
    )(h, parts1, parts1, parts2, parts2, pre, w1h, w1a, w2,
      nb2.reshape(1, H), g.reshape(1, H), b.reshape(1, H))


def kernel(h, edge_index, edge_relation, node_momentum_signature, node_role,
           node_mass_features, edge_channel, rel_emb, role_emb, channel_emb,
           eW1, eb1, eW2, eb2, nW1, nb1, nW2, nb2, ln_g, ln_b):
    N = h.shape[0]
    E = edge_index.shape[1]
    E2 = E // 2
    sig = node_momentum_signature
    mass = node_mass_features
    src, dst = edge_index[0], edge_index[1]

    roh = jax.nn.one_hot(node_role, 6, dtype=jnp.float32)
    smr = jnp.concatenate(
        [sig, mass, roh, jnp.ones((N, 1), jnp.float32),
         jnp.zeros((N, 3), jnp.float32)], axis=-1)
    zpad = jnp.zeros((4, H), jnp.float32)
    ws = jnp.concatenate([eW1[256:260], eW1[276:278],
                          role_emb @ eW1[296:304], zpad], axis=0)
    wd = jnp.concatenate([eW1[260:264], eW1[278:280],
                          role_emb @ eW1[304:312], zpad], axis=0)
    wp = jnp.concatenate([nW1[256:260], nW1[260:262],
                          role_emb @ nW1[262:270], nb1.reshape(1, H),
                          jnp.zeros((3, H), jnp.float32)], axis=0)
    RelP = rel_emb @ eW1[280:296] + eb1
    ChP = channel_emb @ eW1[312:320]
    w24 = jnp.concatenate([eW1[264:276], RelP, ChP], axis=0)
    sigp = jnp.concatenate([sig, jnp.zeros((N, 12), jnp.float32)], axis=-1)
    relf = edge_relation.astype(jnp.float32).reshape(-1, 1)
    chf = edge_channel.astype(jnp.float32).reshape(-1, 1)
    zrows = jnp.zeros((_NP, H), jnp.float32)

    asrc, adst, pre = _pre_kernel(h, smr, eW1[0:128], eW1[128:256], ws, wd, wp)

    gather_rows = _make_ring_gather(E2, H, 200, untiled=False)
    gather_sigs = _make_ring_gather(E2, 16, 1000, untiled=True)
    scatter = _make_scatter(E2, 200)

    halves = []
    for k in range(2):
        sl = slice(k * E2, (k + 1) * E2)
        srk, dsk = src[sl], dst[sl]
        ga, gb = gather_rows(srk, dsk, asrc, adst)
        ss, sd = gather_sigs(srk, dsk, sigp, sigp)
        halves.append((srk, dsk, ga, gb, ss, sd, relf[sl], chf[sl]))

    parts = []
    for k in range(2):
        srk, dsk, ga, gb, ss, sd, rf, cf = halves[k]
        m = _edge_mlp(ga, gb, ss, sd, rf, cf, w24, eW2, eb2)
        parts.append(scatter(m, dsk, zrows))

    return _node_mlp(h, parts[0], parts[1], pre, nW1[0:128],
                     nW1[128:256], nW2, nb2, ln_g, ln_b)
